# baseline (speedup 1.0000x reference)
import functools
import jax
import jax.numpy as jnp
from jax.experimental import pallas as pl
from jax.experimental.pallas import tpu as pltpu
from jax.experimental.pallas import tpu_sc as plsc

B, N = 4, 1024
DIM, POS_DIM, K = 256, 16, 16
ALPHA, SIGMA = 0.1, 1.0

_NC, _NS, _L = 2, 16, 16
_NW = _NC * _NS
_RPW = (B * N) // _NW
_GR = 16


def _tc1_kernel(states_ref, positions_ref, wf_ref, bf_ref, wv_ref, bv_ref,
                sq_ref, newpos_ref, flow_ref, values_ref):
    states = states_ref[0]
    positions = positions_ref[0]
    flow = jax.lax.dot_general(
        states, wf_ref[...], (((1,), (1,)), ((), ())),
        preferred_element_type=jnp.float32) + bf_ref[...][None, :]
    newpos = positions + ALPHA * flow
    flow_ref[0] = flow
    newpos_ref[0] = newpos
    values_ref[0] = jax.lax.dot_general(
        states, wv_ref[...], (((1,), (1,)), ((), ())),
        preferred_element_type=jnp.float32) + bv_ref[...][None, :]
    gram = jax.lax.dot_general(
        newpos, newpos, (((1,), (1,)), ((), ())),
        precision=jax.lax.Precision.HIGHEST,
        preferred_element_type=jnp.float32)
    sqn = jnp.sum(newpos * newpos, axis=1, keepdims=True)
    ones_row = jnp.ones((1, POS_DIM), dtype=jnp.float32)
    sqn_cols = jax.lax.dot_general(
        ones_row, newpos * newpos, (((1,), (1,)), ((), ())),
        precision=jax.lax.Precision.HIGHEST,
        preferred_element_type=jnp.float32)
    sq_ref[0] = jnp.maximum(sqn + sqn_cols - 2.0 * gram, 0.0)


def _sc_thresh_kernel(sq_hbm, t_hbm, rows_v, tvec_v):
    wid = jax.lax.axis_index("s") * _NC + jax.lax.axis_index("c")
    base = wid * _RPW
    iota = jax.lax.iota(jnp.int32, _L)
    big = jnp.float32(3e38)

    for g in range(_RPW // _GR):
        row0 = base + g * _GR
        pltpu.sync_copy(sq_hbm.at[pl.ds(row0, _GR)], rows_v)

        def row_body(r, tvec):
            rg = row0 + r
            col = rg % N
            cd = col // _L
            ln = col % _L

            def chunk_body(c, cur):
                v = rows_v[r, pl.ds(c * _L, _L)]
                v = jnp.where(jnp.logical_and(iota == ln, c == cd), big, v)
                vs = jnp.sort(v)
                lo = jnp.minimum(vs, jax.lax.rev(cur, (0,)))
                return jnp.sort(lo)

            cur = jax.lax.fori_loop(
                0, N // _L, chunk_body, jnp.full((_L,), big, jnp.float32))
            return jnp.where(iota == r, cur[_L - 1], tvec)

        tvec_v[...] = jax.lax.fori_loop(
            0, _GR, row_body, jnp.zeros((_L,), jnp.float32))
        pltpu.sync_copy(tvec_v, t_hbm.at[pl.ds(row0, _GR)])


def _sc_thresh(sq_flat):
    mesh = plsc.VectorSubcoreMesh(core_axis_name="c", subcore_axis_name="s")
    return pl.kernel(
        _sc_thresh_kernel,
        mesh=mesh,
        out_type=jax.ShapeDtypeStruct((B * N,), jnp.float32),
        scratch_types=[
            pltpu.VMEM((_GR, N), jnp.float32),
            pltpu.VMEM((_L,), jnp.float32),
        ],
        compiler_params=pltpu.CompilerParams(needs_layout_passes=False),
    )(sq_flat)


def _tc2_kernel(sq_ref, t_ref, values_ref, ctx_ref):
    sqm = sq_ref[0]
    t = t_ref[0]
    values = values_ref[0]
    iota_j = jax.lax.broadcasted_iota(jnp.int32, (N, N), 1)
    iota_i = jax.lax.broadcasted_iota(jnp.int32, (N, N), 0)
    d = jnp.sqrt(sqm)
    w = jnp.where((sqm <= t) & (iota_j != iota_i),
                  jnp.exp(d * (-1.0 / (2.0 * SIGMA ** 2))), 0.0)
    s = jax.lax.dot_general(
        w, jnp.ones((N, 1), dtype=jnp.float32), (((1,), (0,)), ((), ())),
        preferred_element_type=jnp.float32) + 1e-8
    ctx_ref[0] = jax.lax.dot_general(
        w, values, (((1,), (0,)), ((), ())),
        preferred_element_type=jnp.float32) / s


def kernel(states, positions, W_flow, b_flow, W_val, b_val):
    sq, new_positions, flow_vectors, values = pl.pallas_call(
        _tc1_kernel,
        grid=(B,),
        in_specs=[
            pl.BlockSpec((1, N, DIM), lambda b: (b, 0, 0)),
            pl.BlockSpec((1, N, POS_DIM), lambda b: (b, 0, 0)),
            pl.BlockSpec((POS_DIM, DIM), lambda b: (0, 0)),
            pl.BlockSpec((POS_DIM,), lambda b: (0,)),
            pl.BlockSpec((DIM, DIM), lambda b: (0, 0)),
            pl.BlockSpec((DIM,), lambda b: (0,)),
        ],
        out_specs=(
            pl.BlockSpec((1, N, N), lambda b: (b, 0, 0)),
            pl.BlockSpec((1, N, POS_DIM), lambda b: (b, 0, 0)),
            pl.BlockSpec((1, N, POS_DIM), lambda b: (b, 0, 0)),
            pl.BlockSpec((1, N, DIM), lambda b: (b, 0, 0)),
        ),
        out_shape=(
            jax.ShapeDtypeStruct((B, N, N), jnp.float32),
            jax.ShapeDtypeStruct((B, N, POS_DIM), jnp.float32),
            jax.ShapeDtypeStruct((B, N, POS_DIM), jnp.float32),
            jax.ShapeDtypeStruct((B, N, DIM), jnp.float32),
        ),
    )(states, positions, W_flow, b_flow, W_val, b_val)

    t = _sc_thresh(sq.reshape(B * N, N)).reshape(B, N, 1)

    context = pl.pallas_call(
        _tc2_kernel,
        grid=(B,),
        in_specs=[
            pl.BlockSpec((1, N, N), lambda b: (b, 0, 0)),
            pl.BlockSpec((1, N, 1), lambda b: (b, 0, 0)),
            pl.BlockSpec((1, N, DIM), lambda b: (b, 0, 0)),
        ],
        out_specs=pl.BlockSpec((1, N, DIM), lambda b: (b, 0, 0)),
        out_shape=jax.ShapeDtypeStruct((B, N, DIM), jnp.float32),
    )(sq, t, values)
    return (context, new_positions, flow_vectors)


# speedup vs baseline: 2.3125x; 2.3125x over previous
import jax
import jax.numpy as jnp
from jax.experimental import pallas as pl
from jax.experimental.pallas import tpu as pltpu

B, N = 4, 1024
DIM, POS_DIM, K = 256, 16, 16
ALPHA, SIGMA = 0.1, 1.0


def _fused_kernel(states_ref, positions_ref, wf_ref, bf_ref, wv_ref, bv_ref,
                  ctx_ref, newpos_ref, flow_ref, dsel_ref):
    states = states_ref[0]
    positions = positions_ref[0]

    flow = jax.lax.dot_general(
        states, wf_ref[...],
        (((1,), (1,)), ((), ())),
        preferred_element_type=jnp.float32) + bf_ref[...][None, :]
    newpos = positions + ALPHA * flow
    flow_ref[0] = flow
    newpos_ref[0] = newpos

    values = jax.lax.dot_general(
        states, wv_ref[...],
        (((1,), (1,)), ((), ())),
        preferred_element_type=jnp.float32) + bv_ref[...][None, :]

    gram = jax.lax.dot_general(
        newpos, newpos,
        (((1,), (1,)), ((), ())),
        precision=jax.lax.Precision.HIGHEST,
        preferred_element_type=jnp.float32)
    sqn = jnp.sum(newpos * newpos, axis=1, keepdims=True)
    ones_row = jnp.ones((1, POS_DIM), dtype=jnp.float32)
    sqn_cols = jax.lax.dot_general(
        ones_row, newpos * newpos,
        (((1,), (1,)), ((), ())),
        precision=jax.lax.Precision.HIGHEST,
        preferred_element_type=jnp.float32)
    sq = jnp.maximum(sqn + sqn_cols - 2.0 * gram, 0.0)

    dsel_ref[...] = sq

    big = jnp.float32(3e38)

    def two_min_pass(m_prev):
        def slab_body(it, carry):
            m1, m2 = carry
            for u in range(4):
                start = pl.multiple_of(it * 32 + u * 8, 8)
                slab = dsel_ref[pl.ds(start, 8), :]
                x = jnp.where(slab > m_prev, slab, big)
                m2 = jnp.minimum(m2, jnp.maximum(m1, x))
                m1 = jnp.minimum(m1, x)
            return m1, m2

        m1, m2 = jax.lax.fori_loop(
            0, N // 32, slab_body,
            (jnp.full((8, N), big, jnp.float32),
             jnp.full((8, N), big, jnp.float32)))
        while m1.shape[0] > 1:
            h = m1.shape[0] // 2
            a1, b1 = m1[:h], m1[h:]
            a2, b2 = m2[:h], m2[h:]
            m1, m2 = (jnp.minimum(a1, b1),
                      jnp.minimum(jnp.maximum(a1, b1), jnp.minimum(a2, b2)))
        return m2

    def body(_, m_prev):
        return two_min_pass(m_prev)

    m16 = jax.lax.fori_loop(
        0, (K + 1) // 2, body, jnp.full((1, N), -1.0, dtype=jnp.float32))
    t = jnp.min(jnp.where(dsel_ref[...] > m16, dsel_ref[...], big),
                axis=0, keepdims=True)

    iota_j = jax.lax.broadcasted_iota(jnp.int32, (N, N), 1)
    iota_i = jax.lax.broadcasted_iota(jnp.int32, (N, N), 0)
    d = jnp.sqrt(sq)
    wT = jnp.where((sq <= t) & (iota_j != iota_i),
                   jnp.exp(d * (-1.0 / (2.0 * SIGMA ** 2))), 0.0)
    s = jax.lax.dot_general(
        wT, jnp.ones((N, 1), dtype=jnp.float32),
        (((0,), (0,)), ((), ())),
        preferred_element_type=jnp.float32) + 1e-8
    ctx = jax.lax.dot_general(
        wT, values,
        (((0,), (0,)), ((), ())),
        preferred_element_type=jnp.float32) / s
    ctx_ref[0] = ctx


def kernel(states, positions, W_flow, b_flow, W_val, b_val):
    grid = (B,)
    out_shapes = (
        jax.ShapeDtypeStruct((B, N, DIM), jnp.float32),
        jax.ShapeDtypeStruct((B, N, POS_DIM), jnp.float32),
        jax.ShapeDtypeStruct((B, N, POS_DIM), jnp.float32),
    )
    in_specs = [
        pl.BlockSpec((1, N, DIM), lambda b: (b, 0, 0)),
        pl.BlockSpec((1, N, POS_DIM), lambda b: (b, 0, 0)),
        pl.BlockSpec((POS_DIM, DIM), lambda b: (0, 0)),
        pl.BlockSpec((POS_DIM,), lambda b: (0,)),
        pl.BlockSpec((DIM, DIM), lambda b: (0, 0)),
        pl.BlockSpec((DIM,), lambda b: (0,)),
    ]
    out_specs = (
        pl.BlockSpec((1, N, DIM), lambda b: (b, 0, 0)),
        pl.BlockSpec((1, N, POS_DIM), lambda b: (b, 0, 0)),
        pl.BlockSpec((1, N, POS_DIM), lambda b: (b, 0, 0)),
    )
    context, new_positions, flow_vectors = pl.pallas_call(
        _fused_kernel,
        grid=grid,
        in_specs=in_specs,
        out_specs=out_specs,
        out_shape=out_shapes,
        scratch_shapes=[
            pltpu.VMEM((N, N), jnp.float32),
        ],
    )(states, positions, W_flow, b_flow, W_val, b_val)
    return (context, new_positions, flow_vectors)
